# baseline (speedup 1.0000x reference)
import functools
import jax
import jax.numpy as jnp
from jax import lax
from jax.experimental import pallas as pl
from jax.experimental.pallas import tpu as pltpu
from jax.experimental.pallas import tpu_sc as plsc

N = 10000
E = 320000
D = 128
NUM_LAYERS = 3
EPS = 1e-5

NW = 32
RPS = 313
SENT = RPS
MAGIC = 13401
MSHIFT = 22

CHUNK = 2000
NCHUNKS = E // CHUNK
STG = 4096
FLUSH = 2048
GS = 128
SB = 8192
EPAD = E + SB + 256

BR = 2000


def _tc_in_body(x_ref, wp_ref, bp_ref, ws_ref, h_ref, s_ref):
    x = x_ref[...]
    h = jnp.dot(x, wp_ref[...].T, preferred_element_type=jnp.float32)
    h_ref[...] = jnp.maximum(h + bp_ref[...], 0.0)
    s_ref[...] = jnp.dot(x, ws_ref[...].T, preferred_element_type=jnp.float32)


def _tc_in(x, wp, bp, ws):
    return pl.pallas_call(
        _tc_in_body,
        grid=(N // BR,),
        in_specs=[
            pl.BlockSpec((BR, D), lambda i: (i, 0)),
            pl.BlockSpec((D, D), lambda i: (0, 0)),
            pl.BlockSpec((1, D), lambda i: (0, 0)),
            pl.BlockSpec((D, D), lambda i: (0, 0)),
        ],
        out_specs=[
            pl.BlockSpec((BR, D), lambda i: (i, 0)),
            pl.BlockSpec((BR, D), lambda i: (i, 0)),
        ],
        out_shape=[
            jax.ShapeDtypeStruct((N, D), jnp.float32),
            jax.ShapeDtypeStruct((N, D), jnp.float32),
        ],
    )(x, wp, bp.reshape(1, D), ws)


def _norm_elu(op, mu, var, gamma, beta):
    inv = lax.rsqrt(var + EPS)
    xn = (op - mu) * inv * gamma + beta
    return jnp.where(xn > 0.0, xn, jnp.exp(jnp.minimum(xn, 0.0)) - 1.0)


def _tc_in_fused_body(op_ref, st_ref, g_ref, b_ref, wp_ref, bp_ref, ws_ref,
                      h_ref, s_ref):
    st = st_ref[...]
    mu = st[0:1, :] / N
    var = st[1:2, :] / N - mu * mu
    x = _norm_elu(op_ref[...], mu, var, g_ref[...], b_ref[...])
    h = jnp.dot(x, wp_ref[...].T, preferred_element_type=jnp.float32)
    h_ref[...] = jnp.maximum(h + bp_ref[...], 0.0)
    s_ref[...] = jnp.dot(x, ws_ref[...].T, preferred_element_type=jnp.float32)


def _tc_in_fused(op, st, gamma, beta, wp, bp, ws):
    return pl.pallas_call(
        _tc_in_fused_body,
        grid=(N // BR,),
        in_specs=[
            pl.BlockSpec((BR, D), lambda i: (i, 0)),
            pl.BlockSpec((2, D), lambda i: (0, 0)),
            pl.BlockSpec((1, D), lambda i: (0, 0)),
            pl.BlockSpec((1, D), lambda i: (0, 0)),
            pl.BlockSpec((D, D), lambda i: (0, 0)),
            pl.BlockSpec((1, D), lambda i: (0, 0)),
            pl.BlockSpec((D, D), lambda i: (0, 0)),
        ],
        out_specs=[
            pl.BlockSpec((BR, D), lambda i: (i, 0)),
            pl.BlockSpec((BR, D), lambda i: (i, 0)),
        ],
        out_shape=[
            jax.ShapeDtypeStruct((N, D), jnp.float32),
            jax.ShapeDtypeStruct((N, D), jnp.float32),
        ],
    )(op, st, gamma.reshape(1, D), beta.reshape(1, D), wp, bp.reshape(1, D), ws)


def _tc_out_body(s_ref, ng_ref, wn_ref, b_ref, op_ref, st_ref):
    i = pl.program_id(0)
    nb = jnp.dot(ng_ref[...], wn_ref[...].T, preferred_element_type=jnp.float32)
    o = s_ref[...] + nb + b_ref[...]
    op_ref[...] = o

    @pl.when(i == 0)
    def _():
        st_ref[...] = jnp.zeros((2, D), jnp.float32)

    ps = jnp.sum(o, axis=0, keepdims=True)
    pss = jnp.sum(o * o, axis=0, keepdims=True)
    st_ref[...] += jnp.concatenate([ps, pss], axis=0)


def _tc_out(s, neigh, wn, b):
    return pl.pallas_call(
        _tc_out_body,
        grid=(N // BR,),
        in_specs=[
            pl.BlockSpec((BR, D), lambda i: (i, 0)),
            pl.BlockSpec((BR, D), lambda i: (i, 0)),
            pl.BlockSpec((D, D), lambda i: (0, 0)),
            pl.BlockSpec((1, D), lambda i: (0, 0)),
        ],
        out_specs=[
            pl.BlockSpec((BR, D), lambda i: (i, 0)),
            pl.BlockSpec((2, D), lambda i: (0, 0)),
        ],
        out_shape=[
            jax.ShapeDtypeStruct((N, D), jnp.float32),
            jax.ShapeDtypeStruct((2, D), jnp.float32),
        ],
    )(s, neigh, wn, b.reshape(1, D))


def _tc_final_body(op_ref, st_ref, g_ref, b_ref, out_ref):
    st = st_ref[...]
    mu = st[0:1, :] / N
    var = st[1:2, :] / N - mu * mu
    out_ref[...] = _norm_elu(op_ref[...], mu, var, g_ref[...], b_ref[...])


def _tc_final(op, st, gamma, beta):
    return pl.pallas_call(
        _tc_final_body,
        grid=(N // BR,),
        in_specs=[
            pl.BlockSpec((BR, D), lambda i: (i, 0)),
            pl.BlockSpec((2, D), lambda i: (0, 0)),
            pl.BlockSpec((1, D), lambda i: (0, 0)),
            pl.BlockSpec((1, D), lambda i: (0, 0)),
        ],
        out_specs=pl.BlockSpec((BR, D), lambda i: (i, 0)),
        out_shape=jax.ShapeDtypeStruct((N, D), jnp.float32),
    )(op, st, gamma.reshape(1, D), beta.reshape(1, D))


_MESH = plsc.VectorSubcoreMesh(core_axis_name="c", subcore_axis_name="s",
                               num_cores=2, num_subcores=16)


def _wid():
    return lax.axis_index("s") * 2 + lax.axis_index("c")


def _take(x, idx):
    return lax.gather(
        x, idx[:, None],
        lax.GatherDimensionNumbers(offset_dims=(), collapsed_slice_dims=(0,),
                                   start_index_map=(0,)),
        (1,), mode=lax.GatherScatterMode.PROMISE_IN_BOUNDS)


def _bucket_body(src_hbm, dst_hbm, w_hbm,
                 bsrc, bdst, bw, bcnt,
                 src_a, dst_a, w_a, src_b, dst_b, w_b,
                 st_src, st_dst, st_w, cnt_v, sem_a, sem_b):
    wid = _wid()
    A = (src_a, dst_a, w_a)
    B = (src_b, dst_b, w_b)

    def issue(c, bufs, sem):
        cbase = pl.multiple_of(c * CHUNK, 8)
        pltpu.async_copy(src_hbm.at[pl.ds(cbase, CHUNK)], bufs[0], sem)
        pltpu.async_copy(dst_hbm.at[pl.ds(cbase, CHUNK)], bufs[1], sem)
        pltpu.async_copy(w_hbm.at[pl.ds(cbase, CHUNK)], bufs[2], sem)

    def wait(bufs, sem):
        pltpu.make_async_copy(src_hbm.at[pl.ds(0, CHUNK)], bufs[0], sem).wait()
        pltpu.make_async_copy(dst_hbm.at[pl.ds(0, CHUNK)], bufs[1], sem).wait()
        pltpu.make_async_copy(w_hbm.at[pl.ds(0, CHUNK)], bufs[2], sem).wait()

    def flush(written, cnt):
        do = cnt >= FLUSH

        @pl.when(do)
        def _():
            base = pl.multiple_of(wid * EPAD + written, 8)
            pltpu.sync_copy(st_src.at[pl.ds(0, FLUSH)],
                            bsrc.at[pl.ds(base, FLUSH)])
            pltpu.sync_copy(st_dst.at[pl.ds(0, FLUSH)],
                            bdst.at[pl.ds(base, FLUSH)])
            pltpu.sync_copy(st_w.at[pl.ds(0, FLUSH)],
                            bw.at[pl.ds(base, FLUSH)])

            def shift(j, _):
                s = pl.ds(FLUSH + j * 16, 16)
                t = pl.ds(j * 16, 16)
                st_src[t] = st_src[s]
                st_dst[t] = st_dst[s]
                st_w[t] = st_w[s]
                return 0

            lax.fori_loop(0, (STG - FLUSH) // 16, shift, 0)

        written = jnp.where(do, written + FLUSH, written)
        cnt = jnp.where(do, cnt - FLUSH, cnt)
        return written, cnt

    def compact(bufs, carry):
        written, cnt = carry
        sc, dc, wc = bufs

        def vec_body(i, cntv):
            sl = pl.ds(i * 16, 16)
            d = dc[sl]
            b = (d * MAGIC) >> MSHIFT
            m = b == wid
            dl = d - b * RPS
            mi = m.astype(jnp.int32)
            pref = plsc.cumsum(mi)
            pos = cntv + pref - mi
            plsc.store_scatter(st_src, [pos], sc[sl], mask=m)
            plsc.store_scatter(st_dst, [pos], dl, mask=m)
            plsc.store_scatter(st_w, [pos], wc[sl], mask=m)
            return cntv + plsc.all_reduce_population_count(m)

        cntv = lax.fori_loop(0, CHUNK // 16, vec_body,
                             jnp.full((16,), cnt, jnp.int32))
        return flush(written, cntv[0])

    issue(0, A, sem_a)

    def pair_body(p, carry):
        issue(2 * p + 1, B, sem_b)
        wait(A, sem_a)
        carry = compact(A, carry)

        @pl.when(p + 1 < NCHUNKS // 2)
        def _():
            issue(2 * p + 2, A, sem_a)

        wait(B, sem_b)
        carry = compact(B, carry)
        return carry

    written, cnt = lax.fori_loop(0, NCHUNKS // 2, pair_body,
                                 (jnp.int32(0), jnp.int32(0)))

    for k in range(16):
        sl = pl.ds(cnt + k * 16, 16)
        st_src[sl] = jnp.zeros((16,), jnp.int32)
        st_dst[sl] = jnp.full((16,), SENT, jnp.int32)
        st_w[sl] = jnp.zeros((16,), jnp.float32)
    cnt = ((cnt + 255) >> 8) << 8

    written, cnt = flush(written, cnt)
    base = pl.multiple_of(wid * EPAD + written, 8)
    pltpu.sync_copy(st_src.at[pl.ds(0, FLUSH)],
                    bsrc.at[pl.ds(base, FLUSH)])
    pltpu.sync_copy(st_dst.at[pl.ds(0, FLUSH)],
                    bdst.at[pl.ds(base, FLUSH)])
    pltpu.sync_copy(st_w.at[pl.ds(0, FLUSH)],
                    bw.at[pl.ds(base, FLUSH)])
    total = written + cnt
    cnt_v[...] = jnp.full((16,), total, jnp.int32)
    pltpu.sync_copy(cnt_v, bcnt.at[pl.ds(pl.multiple_of(wid * 16, 16), 16)])


_bucket = pl.kernel(
    _bucket_body,
    out_type=(
        jax.ShapeDtypeStruct((NW * EPAD,), jnp.int32),
        jax.ShapeDtypeStruct((NW * EPAD,), jnp.int32),
        jax.ShapeDtypeStruct((NW * EPAD,), jnp.float32),
        jax.ShapeDtypeStruct((NW * 16,), jnp.int32),
    ),
    mesh=_MESH,
    compiler_params=pltpu.CompilerParams(needs_layout_passes=False),
    scratch_types=[
        pltpu.VMEM((CHUNK,), jnp.int32),
        pltpu.VMEM((CHUNK,), jnp.int32),
        pltpu.VMEM((CHUNK,), jnp.float32),
        pltpu.VMEM((CHUNK,), jnp.int32),
        pltpu.VMEM((CHUNK,), jnp.int32),
        pltpu.VMEM((CHUNK,), jnp.float32),
        pltpu.VMEM((STG,), jnp.int32),
        pltpu.VMEM((STG,), jnp.int32),
        pltpu.VMEM((STG,), jnp.float32),
        pltpu.VMEM((16,), jnp.int32),
        pltpu.SemaphoreType.DMA,
        pltpu.SemaphoreType.DMA,
    ],
)


def _segmax_body(h_hbm, bsrc, bdst, bw, bcnt,
                 out_hbm,
                 acc, msrc, mdst, mw, rows_a, rows_b, cnt_v, sem_a, sem_b):
    wid = _wid()
    iota = lax.iota(jnp.int32, 16)

    def zero_body(r, _):
        acc[pl.ds(r * 16, 16)] = jnp.zeros((16,), jnp.float32)
        return 0

    lax.fori_loop(0, (RPS + 1) * D // 16, zero_body, 0)

    pltpu.sync_copy(bcnt.at[pl.ds(pl.multiple_of(wid * 16, 16), 16)], cnt_v)
    cnt = cnt_v[...][0]
    ebase = wid * EPAD
    nsb = (cnt + SB - 1) >> 13

    def issue(g, rows, sem):
        pltpu.async_copy(h_hbm.at[msrc.at[pl.ds(g * GS, GS)]], rows, sem)

    def wait(rows, sem):
        pltpu.make_async_copy(h_hbm.at[pl.ds(0, GS)], rows, sem).wait()

    def process(g, rows):
        gb = g * GS

        def blk(b, _):
            sl = pl.ds(gb + b * 16, 16)
            dvec = mdst[sl]
            wvec = mw[sl]
            srt, perm = plsc.sort_key_val(dvec, iota)
            shiftidx = jnp.maximum(iota - 1, 0)
            prev = _take(srt, shiftidx)
            dupm = (srt == prev) & (iota > 0)
            dup_i32 = dupm.astype(jnp.int32)
            run3 = dup_i32 * _take(dup_i32, shiftidx)
            has3 = plsc.all_reduce_population_count(run3 > 0)[0] > 0

            @pl.when(jnp.logical_not(has3))
            def _():
                wsrt = _take(wvec, perm)
                nd = _take(dup_i32, jnp.minimum(iota + 1, 15))
                scat = (nd == 0) | (iota == 15)
                rp = perm + b * 16
                idx = srt * D
                for f in range(D):
                    cur = plsc.load_gather(acc, [idx])
                    cf = jnp.full((16,), f, jnp.int32)
                    mv = plsc.load_gather(rows, [rp, cf])
                    val = mv * wsrt
                    pv = _take(val, shiftidx)
                    mg = jnp.maximum(val, jnp.where(dupm, pv, 0.0))
                    mg = jnp.maximum(cur, mg)
                    plsc.store_scatter(acc, [idx], mg, mask=scat)
                    idx = idx + 1

            @pl.when(has3)
            def _():
                for i in range(16):
                    d = dvec[i]
                    wv = wvec[i]
                    for f in range(D // 16):
                        fs = pl.ds(d * D + f * 16, 16)
                        acc[fs] = jnp.maximum(
                            acc[fs], rows[b * 16 + i, pl.ds(f * 16, 16)] * wv)
            return 0

        lax.fori_loop(0, GS // 16, blk, 0)

    def sb_body(sb, _):
        mbase = pl.multiple_of(ebase + sb * SB, 8)
        pltpu.sync_copy(bsrc.at[pl.ds(mbase, SB)], msrc)
        pltpu.sync_copy(bdst.at[pl.ds(mbase, SB)], mdst)
        pltpu.sync_copy(bw.at[pl.ds(mbase, SB)], mw)
        rem = cnt - sb * SB
        npairs = jnp.minimum(rem, SB) >> 8

        issue(0, rows_a, sem_a)

        def pair_body(j, _):
            issue(2 * j + 1, rows_b, sem_b)
            wait(rows_a, sem_a)
            process(2 * j, rows_a)

            @pl.when(j + 1 < npairs)
            def _():
                issue(2 * j + 2, rows_a, sem_a)

            wait(rows_b, sem_b)
            process(2 * j + 1, rows_b)
            return 0

        lax.fori_loop(0, npairs, pair_body, 0)
        return 0

    lax.fori_loop(0, nsb, sb_body, 0)
    pltpu.sync_copy(acc.at[pl.ds(0, RPS * D)], out_hbm.at[wid])


_segmax = pl.kernel(
    _segmax_body,
    out_type=jax.ShapeDtypeStruct((NW, RPS * D), jnp.float32),
    mesh=_MESH,
    compiler_params=pltpu.CompilerParams(needs_layout_passes=False),
    scratch_types=[
        pltpu.VMEM(((RPS + 1) * D,), jnp.float32),
        pltpu.VMEM((SB,), jnp.int32),
        pltpu.VMEM((SB,), jnp.int32),
        pltpu.VMEM((SB,), jnp.float32),
        pltpu.VMEM((GS, D), jnp.float32),
        pltpu.VMEM((GS, D), jnp.float32),
        pltpu.VMEM((16,), jnp.int32),
        pltpu.SemaphoreType.DMA,
        pltpu.SemaphoreType.DMA,
    ],
)


@jax.jit
def kernel(node_weight, edge_index, edge_weight,
           W_pool_0, b_pool_0, W_neigh_0, W_self_0, b_sage_0, gamma_0, beta_0,
           W_pool_1, b_pool_1, W_neigh_1, W_self_1, b_sage_1, gamma_1, beta_1,
           W_pool_2, b_pool_2, W_neigh_2, W_self_2, b_sage_2, gamma_2, beta_2):
    params = [
        (W_pool_0, b_pool_0, W_neigh_0, W_self_0, b_sage_0, gamma_0, beta_0),
        (W_pool_1, b_pool_1, W_neigh_1, W_self_1, b_sage_1, gamma_1, beta_1),
        (W_pool_2, b_pool_2, W_neigh_2, W_self_2, b_sage_2, gamma_2, beta_2),
    ]
    src = edge_index[0]
    dst = edge_index[1]

    bsrc, bdst, bw, bcnt = _bucket(src, dst, edge_weight)

    op, st = None, None
    for i in range(NUM_LAYERS):
        wp, bp, wn, ws, b, gamma, beta = params[i]
        if i == 0:
            h, s = _tc_in(node_weight, wp, bp, ws)
        else:
            h, s = _tc_in_fused(op, st, params[i - 1][5], params[i - 1][6],
                                wp, bp, ws)
        neigh = _segmax(h, bsrc, bdst, bw, bcnt)
        neigh = neigh.reshape(NW * RPS, D)[:N]
        op, st = _tc_out(s, neigh, wn, b)

    return _tc_final(op, st, params[2][5], params[2][6])


# speedup vs baseline: 3.7287x; 3.7287x over previous
import functools
import jax
import jax.numpy as jnp
from jax import lax
from jax.experimental import pallas as pl
from jax.experimental.pallas import tpu as pltpu
from jax.experimental.pallas import tpu_sc as plsc

N = 10000
E = 320000
D = 128
NUM_LAYERS = 3
EPS = 1e-5

NW = 32
RPS = 313
SENT = RPS
MAGIC = 13401
MSHIFT = 22

CHUNK = 2000
NCHUNKS = E // CHUNK
STG = 4096
FLUSH = 2048
GS = 128
SB = 8192
EPAD = E + SB + 256

BR = 2000


def _tc_in_body(x_ref, wp_ref, bp_ref, ws_ref, h_ref, s_ref):
    x = x_ref[...]
    h = jnp.dot(x, wp_ref[...].T, preferred_element_type=jnp.float32)
    h_ref[...] = jnp.maximum(h + bp_ref[...], 0.0)
    s_ref[...] = jnp.dot(x, ws_ref[...].T, preferred_element_type=jnp.float32)


def _tc_in(x, wp, bp, ws):
    return pl.pallas_call(
        _tc_in_body,
        grid=(N // BR,),
        in_specs=[
            pl.BlockSpec((BR, D), lambda i: (i, 0)),
            pl.BlockSpec((D, D), lambda i: (0, 0)),
            pl.BlockSpec((1, D), lambda i: (0, 0)),
            pl.BlockSpec((D, D), lambda i: (0, 0)),
        ],
        out_specs=[
            pl.BlockSpec((BR, D), lambda i: (i, 0)),
            pl.BlockSpec((BR, D), lambda i: (i, 0)),
        ],
        out_shape=[
            jax.ShapeDtypeStruct((N, D), jnp.float32),
            jax.ShapeDtypeStruct((N, D), jnp.float32),
        ],
    )(x, wp, bp.reshape(1, D), ws)


def _norm_elu(op, mu, var, gamma, beta):
    inv = lax.rsqrt(var + EPS)
    xn = (op - mu) * inv * gamma + beta
    return jnp.where(xn > 0.0, xn, jnp.exp(jnp.minimum(xn, 0.0)) - 1.0)


def _tc_in_fused_body(op_ref, st_ref, g_ref, b_ref, wp_ref, bp_ref, ws_ref,
                      h_ref, s_ref):
    st = st_ref[...]
    mu = st[0:1, :] / N
    var = st[1:2, :] / N - mu * mu
    x = _norm_elu(op_ref[...], mu, var, g_ref[...], b_ref[...])
    h = jnp.dot(x, wp_ref[...].T, preferred_element_type=jnp.float32)
    h_ref[...] = jnp.maximum(h + bp_ref[...], 0.0)
    s_ref[...] = jnp.dot(x, ws_ref[...].T, preferred_element_type=jnp.float32)


def _tc_in_fused(op, st, gamma, beta, wp, bp, ws):
    return pl.pallas_call(
        _tc_in_fused_body,
        grid=(N // BR,),
        in_specs=[
            pl.BlockSpec((BR, D), lambda i: (i, 0)),
            pl.BlockSpec((2, D), lambda i: (0, 0)),
            pl.BlockSpec((1, D), lambda i: (0, 0)),
            pl.BlockSpec((1, D), lambda i: (0, 0)),
            pl.BlockSpec((D, D), lambda i: (0, 0)),
            pl.BlockSpec((1, D), lambda i: (0, 0)),
            pl.BlockSpec((D, D), lambda i: (0, 0)),
        ],
        out_specs=[
            pl.BlockSpec((BR, D), lambda i: (i, 0)),
            pl.BlockSpec((BR, D), lambda i: (i, 0)),
        ],
        out_shape=[
            jax.ShapeDtypeStruct((N, D), jnp.float32),
            jax.ShapeDtypeStruct((N, D), jnp.float32),
        ],
    )(op, st, gamma.reshape(1, D), beta.reshape(1, D), wp, bp.reshape(1, D), ws)


def _tc_out_body(s_ref, ng_ref, wn_ref, b_ref, op_ref, st_ref):
    i = pl.program_id(0)
    nb = jnp.dot(ng_ref[...], wn_ref[...].T, preferred_element_type=jnp.float32)
    o = s_ref[...] + nb + b_ref[...]
    op_ref[...] = o

    @pl.when(i == 0)
    def _():
        st_ref[...] = jnp.zeros((2, D), jnp.float32)

    ps = jnp.sum(o, axis=0, keepdims=True)
    pss = jnp.sum(o * o, axis=0, keepdims=True)
    st_ref[...] += jnp.concatenate([ps, pss], axis=0)


def _tc_out(s, neigh, wn, b):
    return pl.pallas_call(
        _tc_out_body,
        grid=(N // BR,),
        in_specs=[
            pl.BlockSpec((BR, D), lambda i: (i, 0)),
            pl.BlockSpec((BR, D), lambda i: (i, 0)),
            pl.BlockSpec((D, D), lambda i: (0, 0)),
            pl.BlockSpec((1, D), lambda i: (0, 0)),
        ],
        out_specs=[
            pl.BlockSpec((BR, D), lambda i: (i, 0)),
            pl.BlockSpec((2, D), lambda i: (0, 0)),
        ],
        out_shape=[
            jax.ShapeDtypeStruct((N, D), jnp.float32),
            jax.ShapeDtypeStruct((2, D), jnp.float32),
        ],
    )(s, neigh, wn, b.reshape(1, D))


def _tc_final_body(op_ref, st_ref, g_ref, b_ref, out_ref):
    st = st_ref[...]
    mu = st[0:1, :] / N
    var = st[1:2, :] / N - mu * mu
    out_ref[...] = _norm_elu(op_ref[...], mu, var, g_ref[...], b_ref[...])


def _tc_final(op, st, gamma, beta):
    return pl.pallas_call(
        _tc_final_body,
        grid=(N // BR,),
        in_specs=[
            pl.BlockSpec((BR, D), lambda i: (i, 0)),
            pl.BlockSpec((2, D), lambda i: (0, 0)),
            pl.BlockSpec((1, D), lambda i: (0, 0)),
            pl.BlockSpec((1, D), lambda i: (0, 0)),
        ],
        out_specs=pl.BlockSpec((BR, D), lambda i: (i, 0)),
        out_shape=jax.ShapeDtypeStruct((N, D), jnp.float32),
    )(op, st, gamma.reshape(1, D), beta.reshape(1, D))


_MESH = plsc.VectorSubcoreMesh(core_axis_name="c", subcore_axis_name="s",
                               num_cores=2, num_subcores=16)


def _wid():
    return lax.axis_index("s") * 2 + lax.axis_index("c")


def _take(x, idx):
    return lax.gather(
        x, idx[:, None],
        lax.GatherDimensionNumbers(offset_dims=(), collapsed_slice_dims=(0,),
                                   start_index_map=(0,)),
        (1,), mode=lax.GatherScatterMode.PROMISE_IN_BOUNDS)


def _bucket_body(src_hbm, dst_hbm, w_hbm,
                 bsrc, bdst, bw, bcnt,
                 src_a, dst_a, w_a, src_b, dst_b, w_b,
                 st_src, st_dst, st_w, cnt_v, sem_a, sem_b):
    wid = _wid()
    A = (src_a, dst_a, w_a)
    B = (src_b, dst_b, w_b)

    def issue(c, bufs, sem):
        cbase = pl.multiple_of(c * CHUNK, 8)
        pltpu.async_copy(src_hbm.at[pl.ds(cbase, CHUNK)], bufs[0], sem)
        pltpu.async_copy(dst_hbm.at[pl.ds(cbase, CHUNK)], bufs[1], sem)
        pltpu.async_copy(w_hbm.at[pl.ds(cbase, CHUNK)], bufs[2], sem)

    def wait(bufs, sem):
        pltpu.make_async_copy(src_hbm.at[pl.ds(0, CHUNK)], bufs[0], sem).wait()
        pltpu.make_async_copy(dst_hbm.at[pl.ds(0, CHUNK)], bufs[1], sem).wait()
        pltpu.make_async_copy(w_hbm.at[pl.ds(0, CHUNK)], bufs[2], sem).wait()

    def flush(written, cnt):
        do = cnt >= FLUSH

        @pl.when(do)
        def _():
            base = pl.multiple_of(wid * EPAD + written, 8)
            pltpu.sync_copy(st_src.at[pl.ds(0, FLUSH)],
                            bsrc.at[pl.ds(base, FLUSH)])
            pltpu.sync_copy(st_dst.at[pl.ds(0, FLUSH)],
                            bdst.at[pl.ds(base, FLUSH)])
            pltpu.sync_copy(st_w.at[pl.ds(0, FLUSH)],
                            bw.at[pl.ds(base, FLUSH)])

            def shift(j, _):
                s = pl.ds(FLUSH + j * 16, 16)
                t = pl.ds(j * 16, 16)
                st_src[t] = st_src[s]
                st_dst[t] = st_dst[s]
                st_w[t] = st_w[s]
                return 0

            lax.fori_loop(0, (STG - FLUSH) // 16, shift, 0)

        written = jnp.where(do, written + FLUSH, written)
        cnt = jnp.where(do, cnt - FLUSH, cnt)
        return written, cnt

    def compact(bufs, carry):
        written, cnt = carry
        sc, dc, wc = bufs

        def vec_body(i, cntv):
            sl = pl.ds(i * 16, 16)
            d = dc[sl]
            b = (d * MAGIC) >> MSHIFT
            m = b == wid
            dl = d - b * RPS
            mi = m.astype(jnp.int32)
            pref = plsc.cumsum(mi)
            pos = cntv + pref - mi
            plsc.store_scatter(st_src, [pos], sc[sl], mask=m)
            plsc.store_scatter(st_dst, [pos], dl, mask=m)
            plsc.store_scatter(st_w, [pos], wc[sl], mask=m)
            return cntv + plsc.all_reduce_population_count(m)

        cntv = lax.fori_loop(0, CHUNK // 16, vec_body,
                             jnp.full((16,), cnt, jnp.int32))
        return flush(written, cntv[0])

    issue(0, A, sem_a)

    def pair_body(p, carry):
        issue(2 * p + 1, B, sem_b)
        wait(A, sem_a)
        carry = compact(A, carry)

        @pl.when(p + 1 < NCHUNKS // 2)
        def _():
            issue(2 * p + 2, A, sem_a)

        wait(B, sem_b)
        carry = compact(B, carry)
        return carry

    written, cnt = lax.fori_loop(0, NCHUNKS // 2, pair_body,
                                 (jnp.int32(0), jnp.int32(0)))

    for k in range(16):
        sl = pl.ds(cnt + k * 16, 16)
        st_src[sl] = jnp.zeros((16,), jnp.int32)
        st_dst[sl] = jnp.full((16,), SENT, jnp.int32)
        st_w[sl] = jnp.zeros((16,), jnp.float32)
    cnt = ((cnt + 255) >> 8) << 8

    written, cnt = flush(written, cnt)
    base = pl.multiple_of(wid * EPAD + written, 8)
    pltpu.sync_copy(st_src.at[pl.ds(0, FLUSH)],
                    bsrc.at[pl.ds(base, FLUSH)])
    pltpu.sync_copy(st_dst.at[pl.ds(0, FLUSH)],
                    bdst.at[pl.ds(base, FLUSH)])
    pltpu.sync_copy(st_w.at[pl.ds(0, FLUSH)],
                    bw.at[pl.ds(base, FLUSH)])
    total = written + cnt
    cnt_v[...] = jnp.full((16,), total, jnp.int32)
    pltpu.sync_copy(cnt_v, bcnt.at[pl.ds(pl.multiple_of(wid * 16, 16), 16)])


_bucket = pl.kernel(
    _bucket_body,
    out_type=(
        jax.ShapeDtypeStruct((NW * EPAD,), jnp.int32),
        jax.ShapeDtypeStruct((NW * EPAD,), jnp.int32),
        jax.ShapeDtypeStruct((NW * EPAD,), jnp.float32),
        jax.ShapeDtypeStruct((NW * 16,), jnp.int32),
    ),
    mesh=_MESH,
    compiler_params=pltpu.CompilerParams(needs_layout_passes=False),
    scratch_types=[
        pltpu.VMEM((CHUNK,), jnp.int32),
        pltpu.VMEM((CHUNK,), jnp.int32),
        pltpu.VMEM((CHUNK,), jnp.float32),
        pltpu.VMEM((CHUNK,), jnp.int32),
        pltpu.VMEM((CHUNK,), jnp.int32),
        pltpu.VMEM((CHUNK,), jnp.float32),
        pltpu.VMEM((STG,), jnp.int32),
        pltpu.VMEM((STG,), jnp.int32),
        pltpu.VMEM((STG,), jnp.float32),
        pltpu.VMEM((16,), jnp.int32),
        pltpu.SemaphoreType.DMA,
        pltpu.SemaphoreType.DMA,
    ],
)


def _segmax_body(h_hbm, bsrc, bdst, bw, bcnt,
                 out_hbm,
                 acc, msrc, mdst, mw, rows_a, rows_b, cnt_v, sem_a, sem_b):
    wid = _wid()
    iota = lax.iota(jnp.int32, 16)

    def zero_body(r, _):
        acc[pl.ds(r * 16, 16)] = jnp.zeros((16,), jnp.float32)
        return 0

    lax.fori_loop(0, (RPS + 1) * D // 16, zero_body, 0)

    pltpu.sync_copy(bcnt.at[pl.ds(pl.multiple_of(wid * 16, 16), 16)], cnt_v)
    cnt = cnt_v[...][0]
    ebase = wid * EPAD
    nsb = (cnt + SB - 1) >> 13

    def issue(g, rows, sem):
        pltpu.async_copy(h_hbm.at[msrc.at[pl.ds(g * GS, GS)]], rows, sem)

    def wait(rows, sem):
        pltpu.make_async_copy(h_hbm.at[pl.ds(0, GS)], rows, sem).wait()

    def process(g, rows):
        gb = g * GS

        def blk(b, _):
            sl = pl.ds(gb + b * 16, 16)
            dvec = mdst[sl]
            wvec = mw[sl]
            ds_ = [dvec[i] for i in range(16)]
            wb_ = [_take(wvec, jnp.full((16,), i, jnp.int32)) for i in range(16)]
            for i in range(16):
                d = ds_[i]
                base = d * D
                for f in range(D // 16):
                    fs = pl.ds(base + f * 16, 16)
                    acc[fs] = jnp.maximum(
                        acc[fs], rows[b * 16 + i, pl.ds(f * 16, 16)] * wb_[i])
            return 0

        lax.fori_loop(0, GS // 16, blk, 0)

    def sb_body(sb, _):
        mbase = pl.multiple_of(ebase + sb * SB, 8)
        pltpu.sync_copy(bsrc.at[pl.ds(mbase, SB)], msrc)
        pltpu.sync_copy(bdst.at[pl.ds(mbase, SB)], mdst)
        pltpu.sync_copy(bw.at[pl.ds(mbase, SB)], mw)
        rem = cnt - sb * SB
        npairs = jnp.minimum(rem, SB) >> 8

        issue(0, rows_a, sem_a)

        def pair_body(j, _):
            issue(2 * j + 1, rows_b, sem_b)
            wait(rows_a, sem_a)
            process(2 * j, rows_a)

            @pl.when(j + 1 < npairs)
            def _():
                issue(2 * j + 2, rows_a, sem_a)

            wait(rows_b, sem_b)
            process(2 * j + 1, rows_b)
            return 0

        lax.fori_loop(0, npairs, pair_body, 0)
        return 0

    lax.fori_loop(0, nsb, sb_body, 0)
    pltpu.sync_copy(acc.at[pl.ds(0, RPS * D)], out_hbm.at[wid])


_segmax = pl.kernel(
    _segmax_body,
    out_type=jax.ShapeDtypeStruct((NW, RPS * D), jnp.float32),
    mesh=_MESH,
    compiler_params=pltpu.CompilerParams(needs_layout_passes=False),
    scratch_types=[
        pltpu.VMEM(((RPS + 1) * D,), jnp.float32),
        pltpu.VMEM((SB,), jnp.int32),
        pltpu.VMEM((SB,), jnp.int32),
        pltpu.VMEM((SB,), jnp.float32),
        pltpu.VMEM((GS, D), jnp.float32),
        pltpu.VMEM((GS, D), jnp.float32),
        pltpu.VMEM((16,), jnp.int32),
        pltpu.SemaphoreType.DMA,
        pltpu.SemaphoreType.DMA,
    ],
)


@jax.jit
def kernel(node_weight, edge_index, edge_weight,
           W_pool_0, b_pool_0, W_neigh_0, W_self_0, b_sage_0, gamma_0, beta_0,
           W_pool_1, b_pool_1, W_neigh_1, W_self_1, b_sage_1, gamma_1, beta_1,
           W_pool_2, b_pool_2, W_neigh_2, W_self_2, b_sage_2, gamma_2, beta_2):
    params = [
        (W_pool_0, b_pool_0, W_neigh_0, W_self_0, b_sage_0, gamma_0, beta_0),
        (W_pool_1, b_pool_1, W_neigh_1, W_self_1, b_sage_1, gamma_1, beta_1),
        (W_pool_2, b_pool_2, W_neigh_2, W_self_2, b_sage_2, gamma_2, beta_2),
    ]
    src = edge_index[0]
    dst = edge_index[1]

    bsrc, bdst, bw, bcnt = _bucket(src, dst, edge_weight)

    op, st = None, None
    for i in range(NUM_LAYERS):
        wp, bp, wn, ws, b, gamma, beta = params[i]
        if i == 0:
            h, s = _tc_in(node_weight, wp, bp, ws)
        else:
            h, s = _tc_in_fused(op, st, params[i - 1][5], params[i - 1][6],
                                wp, bp, ws)
        neigh = _segmax(h, bsrc, bdst, bw, bcnt)
        neigh = neigh.reshape(NW * RPS, D)[:N]
        op, st = _tc_out(s, neigh, wn, b)

    return _tc_final(op, st, params[2][5], params[2][6])


# speedup vs baseline: 6.0141x; 1.6129x over previous
import functools
import jax
import jax.numpy as jnp
from jax import lax
from jax.experimental import pallas as pl
from jax.experimental.pallas import tpu as pltpu
from jax.experimental.pallas import tpu_sc as plsc

N = 10000
E = 320000
D = 128
NUM_LAYERS = 3
EPS = 1e-5

NW = 32
RPS = 313
SENT = RPS
MAGIC = 13401
MSHIFT = 22

CHUNK = 2000
NCHUNKS = E // CHUNK
STG = 4096
FLUSH = 2048
GS = 64
SB = 4096
EPAD = E + SB + 256

BR = 2000


def _pack_feats(hb):
    parts = []
    for c in range(4):
        lo = hb[:, c * 32:c * 32 + 16].astype(jnp.uint32)
        hi = hb[:, c * 32 + 16:c * 32 + 32].astype(jnp.uint32)
        parts.append(lo | (hi << 16))
    return jnp.concatenate(parts, axis=1)


def _pack_rows(h):
    hb = lax.bitcast_convert_type(h.astype(jnp.bfloat16), jnp.uint16)
    hb3 = hb.reshape(h.shape[0] // 2, 2, 128)
    a = _pack_feats(hb3[:, 0, :])
    b = _pack_feats(hb3[:, 1, :])
    return lax.bitcast_convert_type(jnp.concatenate([a, b], axis=1),
                                    jnp.float32)


def _tc_in_body(x_ref, wp_ref, bp_ref, ws_ref, h_ref, s_ref):
    x = x_ref[...]
    h = jnp.dot(x, wp_ref[...].T, preferred_element_type=jnp.float32)
    h_ref[...] = _pack_rows(jnp.maximum(h + bp_ref[...], 0.0))
    s_ref[...] = jnp.dot(x, ws_ref[...].T, preferred_element_type=jnp.float32)


def _tc_in(x, wp, bp, ws):
    return pl.pallas_call(
        _tc_in_body,
        grid=(N // BR,),
        in_specs=[
            pl.BlockSpec((BR, D), lambda i: (i, 0)),
            pl.BlockSpec((D, D), lambda i: (0, 0)),
            pl.BlockSpec((1, D), lambda i: (0, 0)),
            pl.BlockSpec((D, D), lambda i: (0, 0)),
        ],
        out_specs=[
            pl.BlockSpec((BR // 2, D), lambda i: (i, 0)),
            pl.BlockSpec((BR, D), lambda i: (i, 0)),
        ],
        out_shape=[
            jax.ShapeDtypeStruct((N // 2, D), jnp.float32),
            jax.ShapeDtypeStruct((N, D), jnp.float32),
        ],
    )(x, wp, bp.reshape(1, D), ws)


def _norm_elu(op, mu, var, gamma, beta):
    inv = lax.rsqrt(var + EPS)
    xn = (op - mu) * inv * gamma + beta
    return jnp.where(xn > 0.0, xn, jnp.exp(jnp.minimum(xn, 0.0)) - 1.0)


def _tc_in_fused_body(op_ref, st_ref, g_ref, b_ref, wp_ref, bp_ref, ws_ref,
                      h_ref, s_ref):
    st = st_ref[...]
    mu = st[0:1, :] / N
    var = st[1:2, :] / N - mu * mu
    x = _norm_elu(op_ref[...], mu, var, g_ref[...], b_ref[...])
    h = jnp.dot(x, wp_ref[...].T, preferred_element_type=jnp.float32)
    h_ref[...] = _pack_rows(jnp.maximum(h + bp_ref[...], 0.0))
    s_ref[...] = jnp.dot(x, ws_ref[...].T, preferred_element_type=jnp.float32)


def _tc_in_fused(op, st, gamma, beta, wp, bp, ws):
    return pl.pallas_call(
        _tc_in_fused_body,
        grid=(N // BR,),
        in_specs=[
            pl.BlockSpec((BR, D), lambda i: (i, 0)),
            pl.BlockSpec((2, D), lambda i: (0, 0)),
            pl.BlockSpec((1, D), lambda i: (0, 0)),
            pl.BlockSpec((1, D), lambda i: (0, 0)),
            pl.BlockSpec((D, D), lambda i: (0, 0)),
            pl.BlockSpec((1, D), lambda i: (0, 0)),
            pl.BlockSpec((D, D), lambda i: (0, 0)),
        ],
        out_specs=[
            pl.BlockSpec((BR // 2, D), lambda i: (i, 0)),
            pl.BlockSpec((BR, D), lambda i: (i, 0)),
        ],
        out_shape=[
            jax.ShapeDtypeStruct((N // 2, D), jnp.float32),
            jax.ShapeDtypeStruct((N, D), jnp.float32),
        ],
    )(op, st, gamma.reshape(1, D), beta.reshape(1, D), wp, bp.reshape(1, D), ws)


def _tc_out_body(s_ref, ng_ref, wn_ref, b_ref, op_ref, st_ref):
    i = pl.program_id(0)
    nb = jnp.dot(ng_ref[...], wn_ref[...].T, preferred_element_type=jnp.float32)
    o = s_ref[...] + nb + b_ref[...]
    op_ref[...] = o

    @pl.when(i == 0)
    def _():
        st_ref[...] = jnp.zeros((2, D), jnp.float32)

    ps = jnp.sum(o, axis=0, keepdims=True)
    pss = jnp.sum(o * o, axis=0, keepdims=True)
    st_ref[...] += jnp.concatenate([ps, pss], axis=0)


def _tc_out(s, neigh, wn, b):
    return pl.pallas_call(
        _tc_out_body,
        grid=(N // BR,),
        in_specs=[
            pl.BlockSpec((BR, D), lambda i: (i, 0)),
            pl.BlockSpec((BR, D), lambda i: (i, 0)),
            pl.BlockSpec((D, D), lambda i: (0, 0)),
            pl.BlockSpec((1, D), lambda i: (0, 0)),
        ],
        out_specs=[
            pl.BlockSpec((BR, D), lambda i: (i, 0)),
            pl.BlockSpec((2, D), lambda i: (0, 0)),
        ],
        out_shape=[
            jax.ShapeDtypeStruct((N, D), jnp.float32),
            jax.ShapeDtypeStruct((2, D), jnp.float32),
        ],
    )(s, neigh, wn, b.reshape(1, D))


def _tc_final_body(op_ref, st_ref, g_ref, b_ref, out_ref):
    st = st_ref[...]
    mu = st[0:1, :] / N
    var = st[1:2, :] / N - mu * mu
    out_ref[...] = _norm_elu(op_ref[...], mu, var, g_ref[...], b_ref[...])


def _tc_final(op, st, gamma, beta):
    return pl.pallas_call(
        _tc_final_body,
        grid=(N // BR,),
        in_specs=[
            pl.BlockSpec((BR, D), lambda i: (i, 0)),
            pl.BlockSpec((2, D), lambda i: (0, 0)),
            pl.BlockSpec((1, D), lambda i: (0, 0)),
            pl.BlockSpec((1, D), lambda i: (0, 0)),
        ],
        out_specs=pl.BlockSpec((BR, D), lambda i: (i, 0)),
        out_shape=jax.ShapeDtypeStruct((N, D), jnp.float32),
    )(op, st, gamma.reshape(1, D), beta.reshape(1, D))


_MESH = plsc.VectorSubcoreMesh(core_axis_name="c", subcore_axis_name="s",
                               num_cores=2, num_subcores=16)


def _wid():
    return lax.axis_index("s") * 2 + lax.axis_index("c")


def _take(x, idx):
    return lax.gather(
        x, idx[:, None],
        lax.GatherDimensionNumbers(offset_dims=(), collapsed_slice_dims=(0,),
                                   start_index_map=(0,)),
        (1,), mode=lax.GatherScatterMode.PROMISE_IN_BOUNDS)


def _bucket_body(src_hbm, dst_hbm, w_hbm,
                 bsrc, bdst, bw, bcnt,
                 src_a, dst_a, w_a, src_b, dst_b, w_b,
                 st_src, st_dst, st_w, cnt_v, sem_a, sem_b):
    wid = _wid()
    A = (src_a, dst_a, w_a)
    B = (src_b, dst_b, w_b)

    def issue(c, bufs, sem):
        cbase = pl.multiple_of(c * CHUNK, 8)
        pltpu.async_copy(src_hbm.at[pl.ds(cbase, CHUNK)], bufs[0], sem)
        pltpu.async_copy(dst_hbm.at[pl.ds(cbase, CHUNK)], bufs[1], sem)
        pltpu.async_copy(w_hbm.at[pl.ds(cbase, CHUNK)], bufs[2], sem)

    def wait(bufs, sem):
        pltpu.make_async_copy(src_hbm.at[pl.ds(0, CHUNK)], bufs[0], sem).wait()
        pltpu.make_async_copy(dst_hbm.at[pl.ds(0, CHUNK)], bufs[1], sem).wait()
        pltpu.make_async_copy(w_hbm.at[pl.ds(0, CHUNK)], bufs[2], sem).wait()

    def flush(written, cnt):
        do = cnt >= FLUSH

        @pl.when(do)
        def _():
            base = pl.multiple_of(wid * EPAD + written, 8)
            pltpu.sync_copy(st_src.at[pl.ds(0, FLUSH)],
                            bsrc.at[pl.ds(base, FLUSH)])
            pltpu.sync_copy(st_dst.at[pl.ds(0, FLUSH)],
                            bdst.at[pl.ds(base, FLUSH)])
            pltpu.sync_copy(st_w.at[pl.ds(0, FLUSH)],
                            bw.at[pl.ds(base, FLUSH)])

            def shift(j, _):
                s = pl.ds(FLUSH + j * 16, 16)
                t = pl.ds(j * 16, 16)
                st_src[t] = st_src[s]
                st_dst[t] = st_dst[s]
                st_w[t] = st_w[s]
                return 0

            lax.fori_loop(0, (STG - FLUSH) // 16, shift, 0)

        written = jnp.where(do, written + FLUSH, written)
        cnt = jnp.where(do, cnt - FLUSH, cnt)
        return written, cnt

    def compact(bufs, carry):
        written, cnt = carry
        sc, dc, wc = bufs

        def vec_body(i, cntv):
            sl = pl.ds(i * 16, 16)
            d = dc[sl]
            b = (d * MAGIC) >> MSHIFT
            m = b == wid
            sv = sc[sl]
            dl = (d - b * RPS) | ((sv & 1) << 14)
            mi = m.astype(jnp.int32)
            pref = plsc.cumsum(mi)
            pos = cntv + pref - mi
            plsc.store_scatter(st_src, [pos], sv >> 1, mask=m)
            plsc.store_scatter(st_dst, [pos], dl, mask=m)
            plsc.store_scatter(st_w, [pos], wc[sl], mask=m)
            return cntv + plsc.all_reduce_population_count(m)

        cntv = lax.fori_loop(0, CHUNK // 16, vec_body,
                             jnp.full((16,), cnt, jnp.int32))
        return flush(written, cntv[0])

    issue(0, A, sem_a)

    def pair_body(p, carry):
        issue(2 * p + 1, B, sem_b)
        wait(A, sem_a)
        carry = compact(A, carry)

        @pl.when(p + 1 < NCHUNKS // 2)
        def _():
            issue(2 * p + 2, A, sem_a)

        wait(B, sem_b)
        carry = compact(B, carry)
        return carry

    written, cnt = lax.fori_loop(0, NCHUNKS // 2, pair_body,
                                 (jnp.int32(0), jnp.int32(0)))

    for k in range(16):
        sl = pl.ds(cnt + k * 16, 16)
        st_src[sl] = jnp.zeros((16,), jnp.int32)
        st_dst[sl] = jnp.full((16,), SENT, jnp.int32)
        st_w[sl] = jnp.zeros((16,), jnp.float32)
    cnt = ((cnt + 255) >> 8) << 8

    written, cnt = flush(written, cnt)
    base = pl.multiple_of(wid * EPAD + written, 8)
    pltpu.sync_copy(st_src.at[pl.ds(0, FLUSH)],
                    bsrc.at[pl.ds(base, FLUSH)])
    pltpu.sync_copy(st_dst.at[pl.ds(0, FLUSH)],
                    bdst.at[pl.ds(base, FLUSH)])
    pltpu.sync_copy(st_w.at[pl.ds(0, FLUSH)],
                    bw.at[pl.ds(base, FLUSH)])
    total = written + cnt
    cnt_v[...] = jnp.full((16,), total, jnp.int32)
    pltpu.sync_copy(cnt_v, bcnt.at[pl.ds(pl.multiple_of(wid * 16, 16), 16)])


_bucket = pl.kernel(
    _bucket_body,
    out_type=(
        jax.ShapeDtypeStruct((NW * EPAD,), jnp.int32),
        jax.ShapeDtypeStruct((NW * EPAD,), jnp.int32),
        jax.ShapeDtypeStruct((NW * EPAD,), jnp.float32),
        jax.ShapeDtypeStruct((NW * 16,), jnp.int32),
    ),
    mesh=_MESH,
    compiler_params=pltpu.CompilerParams(needs_layout_passes=False),
    scratch_types=[
        pltpu.VMEM((CHUNK,), jnp.int32),
        pltpu.VMEM((CHUNK,), jnp.int32),
        pltpu.VMEM((CHUNK,), jnp.float32),
        pltpu.VMEM((CHUNK,), jnp.int32),
        pltpu.VMEM((CHUNK,), jnp.int32),
        pltpu.VMEM((CHUNK,), jnp.float32),
        pltpu.VMEM((STG,), jnp.int32),
        pltpu.VMEM((STG,), jnp.int32),
        pltpu.VMEM((STG,), jnp.float32),
        pltpu.VMEM((16,), jnp.int32),
        pltpu.SemaphoreType.DMA,
        pltpu.SemaphoreType.DMA,
    ],
)


def _segmax_body(h_hbm, bsrc, bdst, bw, bcnt,
                 out_hbm,
                 acc, msrc, mdst, mw, rows_a, rows_b, h_sp, cnt_v,
                 sem_a, sem_b):
    wid = _wid()
    iota = lax.iota(jnp.int32, 16)

    @pl.when(lax.axis_index("s") == 0)
    def _():
        pltpu.sync_copy(h_hbm, h_sp)

    plsc.subcore_barrier()


    def zero_body(r, _):
        acc[pl.ds(r * 16, 16)] = jnp.zeros((16,), jnp.float32)
        return 0

    lax.fori_loop(0, (RPS + 1) * D // 16, zero_body, 0)

    pltpu.sync_copy(bcnt.at[pl.ds(pl.multiple_of(wid * 16, 16), 16)], cnt_v)
    cnt = cnt_v[...][0]
    ebase = wid * EPAD
    nsb = (cnt + SB - 1) >> 12

    def issue(g, rows, sem):
        pltpu.async_copy(h_sp.at[msrc.at[pl.ds(g * GS, GS)]], rows, sem)

    def wait(rows, sem):
        pltpu.make_async_copy(h_sp.at[pl.ds(0, GS)], rows, sem).wait()

    def process(g, rows):
        gb = g * GS

        def blk(b, _):
            sl = pl.ds(gb + b * 16, 16)
            dvec = mdst[sl]
            wvec = mw[sl]
            ds_ = [dvec[i] for i in range(16)]
            wb_ = [_take(wvec, jnp.full((16,), i, jnp.int32)) for i in range(16)]
            for i in range(16):
                v = ds_[i]
                d = v & 16383
                roff = (v >> 14) << 6
                wb = wb_[i]
                base = d * D
                r = b * 16 + i
                for c in range(4):
                    pv = rows[r, pl.ds(roff + c * 16, 16)]
                    pb = plsc.bitcast(pv, jnp.bfloat16)
                    a, b2 = plsc.unpack(pb, format=plsc.PackFormat.INTERLEAVED)
                    fs0 = pl.ds(base + c * 32, 16)
                    fs1 = pl.ds(base + c * 32 + 16, 16)
                    acc[fs0] = jnp.maximum(acc[fs0], a * wb)
                    acc[fs1] = jnp.maximum(acc[fs1], b2 * wb)
            return 0

        lax.fori_loop(0, GS // 16, blk, 0)

    def sb_body(sb, _):
        mbase = pl.multiple_of(ebase + sb * SB, 8)
        pltpu.sync_copy(bsrc.at[pl.ds(mbase, SB)], msrc)
        pltpu.sync_copy(bdst.at[pl.ds(mbase, SB)], mdst)
        pltpu.sync_copy(bw.at[pl.ds(mbase, SB)], mw)
        rem = cnt - sb * SB
        npairs = jnp.minimum(rem, SB) >> 7

        issue(0, rows_a, sem_a)

        def pair_body(j, _):
            issue(2 * j + 1, rows_b, sem_b)
            wait(rows_a, sem_a)
            process(2 * j, rows_a)

            @pl.when(j + 1 < npairs)
            def _():
                issue(2 * j + 2, rows_a, sem_a)

            wait(rows_b, sem_b)
            process(2 * j + 1, rows_b)
            return 0

        lax.fori_loop(0, npairs, pair_body, 0)
        return 0

    lax.fori_loop(0, nsb, sb_body, 0)
    pltpu.sync_copy(acc.at[pl.ds(0, RPS * D)], out_hbm.at[wid])


_segmax = pl.kernel(
    _segmax_body,
    out_type=jax.ShapeDtypeStruct((NW, RPS * D), jnp.float32),
    mesh=_MESH,
    compiler_params=pltpu.CompilerParams(needs_layout_passes=False),
    scratch_types=[
        pltpu.VMEM(((RPS + 1) * D,), jnp.float32),
        pltpu.VMEM((SB,), jnp.int32),
        pltpu.VMEM((SB,), jnp.int32),
        pltpu.VMEM((SB,), jnp.float32),
        pltpu.VMEM((GS, D), jnp.float32),
        pltpu.VMEM((GS, D), jnp.float32),
        pltpu.VMEM_SHARED((N // 2, D), jnp.float32),
        pltpu.VMEM((16,), jnp.int32),
        pltpu.SemaphoreType.DMA,
        pltpu.SemaphoreType.DMA,
    ],
)


@jax.jit
def kernel(node_weight, edge_index, edge_weight,
           W_pool_0, b_pool_0, W_neigh_0, W_self_0, b_sage_0, gamma_0, beta_0,
           W_pool_1, b_pool_1, W_neigh_1, W_self_1, b_sage_1, gamma_1, beta_1,
           W_pool_2, b_pool_2, W_neigh_2, W_self_2, b_sage_2, gamma_2, beta_2):
    params = [
        (W_pool_0, b_pool_0, W_neigh_0, W_self_0, b_sage_0, gamma_0, beta_0),
        (W_pool_1, b_pool_1, W_neigh_1, W_self_1, b_sage_1, gamma_1, beta_1),
        (W_pool_2, b_pool_2, W_neigh_2, W_self_2, b_sage_2, gamma_2, beta_2),
    ]
    src = edge_index[0]
    dst = edge_index[1]

    bsrc, bdst, bw, bcnt = _bucket(src, dst, edge_weight)

    op, st = None, None
    for i in range(NUM_LAYERS):
        wp, bp, wn, ws, b, gamma, beta = params[i]
        if i == 0:
            h, s = _tc_in(node_weight, wp, bp, ws)
        else:
            h, s = _tc_in_fused(op, st, params[i - 1][5], params[i - 1][6],
                                wp, bp, ws)
        neigh = _segmax(h, bsrc, bdst, bw, bcnt)
        neigh = neigh.reshape(NW * RPS, D)[:N]
        op, st = _tc_out(s, neigh, wn, b)

    return _tc_final(op, st, params[2][5], params[2][6])


# speedup vs baseline: 7.2410x; 1.2040x over previous
import functools
import jax
import jax.numpy as jnp
from jax import lax
from jax.experimental import pallas as pl
from jax.experimental.pallas import tpu as pltpu
from jax.experimental.pallas import tpu_sc as plsc

N = 10000
E = 320000
D = 128
NUM_LAYERS = 3
EPS = 1e-5

NW = 32
RPS = 313
SENT = RPS
MAGIC = 13401
MSHIFT = 22

CHUNK = 2000
NCHUNKS = E // CHUNK
STG = 4096
FLUSH = 2048
GS = 64
SB = 4096
EPAD = E + SB + 256

BR = 2000


def _pack_feats(hb):
    parts = []
    for c in range(4):
        lo = hb[:, c * 32:c * 32 + 16].astype(jnp.uint32)
        hi = hb[:, c * 32 + 16:c * 32 + 32].astype(jnp.uint32)
        parts.append(lo | (hi << 16))
    return jnp.concatenate(parts, axis=1)


def _pack_rows(h):
    hb = lax.bitcast_convert_type(h.astype(jnp.bfloat16), jnp.uint16)
    hb3 = hb.reshape(h.shape[0] // 2, 2, 128)
    a = _pack_feats(hb3[:, 0, :])
    b = _pack_feats(hb3[:, 1, :])
    return lax.bitcast_convert_type(jnp.concatenate([a, b], axis=1),
                                    jnp.float32)


def _tc_in_body(x_ref, wp_ref, bp_ref, ws_ref, h_ref, s_ref):
    x = x_ref[...]
    h = jnp.dot(x, wp_ref[...].T, preferred_element_type=jnp.float32)
    h_ref[...] = _pack_rows(jnp.maximum(h + bp_ref[...], 0.0))
    s_ref[...] = jnp.dot(x, ws_ref[...].T, preferred_element_type=jnp.float32)


def _tc_in(x, wp, bp, ws):
    return pl.pallas_call(
        _tc_in_body,
        grid=(N // BR,),
        in_specs=[
            pl.BlockSpec((BR, D), lambda i: (i, 0)),
            pl.BlockSpec((D, D), lambda i: (0, 0)),
            pl.BlockSpec((1, D), lambda i: (0, 0)),
            pl.BlockSpec((D, D), lambda i: (0, 0)),
        ],
        out_specs=[
            pl.BlockSpec((BR // 2, D), lambda i: (i, 0)),
            pl.BlockSpec((BR, D), lambda i: (i, 0)),
        ],
        out_shape=[
            jax.ShapeDtypeStruct((N // 2, D), jnp.float32),
            jax.ShapeDtypeStruct((N, D), jnp.float32),
        ],
    )(x, wp, bp.reshape(1, D), ws)


def _norm_elu(op, mu, var, gamma, beta):
    inv = lax.rsqrt(var + EPS)
    xn = (op - mu) * inv * gamma + beta
    return jnp.where(xn > 0.0, xn, jnp.exp(jnp.minimum(xn, 0.0)) - 1.0)


def _tc_in_fused_body(op_ref, st_ref, g_ref, b_ref, wp_ref, bp_ref, ws_ref,
                      h_ref, s_ref):
    st = st_ref[...]
    mu = st[0:1, :] / N
    var = st[1:2, :] / N - mu * mu
    x = _norm_elu(op_ref[...], mu, var, g_ref[...], b_ref[...])
    h = jnp.dot(x, wp_ref[...].T, preferred_element_type=jnp.float32)
    h_ref[...] = _pack_rows(jnp.maximum(h + bp_ref[...], 0.0))
    s_ref[...] = jnp.dot(x, ws_ref[...].T, preferred_element_type=jnp.float32)


def _tc_in_fused(op, st, gamma, beta, wp, bp, ws):
    return pl.pallas_call(
        _tc_in_fused_body,
        grid=(N // BR,),
        in_specs=[
            pl.BlockSpec((BR, D), lambda i: (i, 0)),
            pl.BlockSpec((2, D), lambda i: (0, 0)),
            pl.BlockSpec((1, D), lambda i: (0, 0)),
            pl.BlockSpec((1, D), lambda i: (0, 0)),
            pl.BlockSpec((D, D), lambda i: (0, 0)),
            pl.BlockSpec((1, D), lambda i: (0, 0)),
            pl.BlockSpec((D, D), lambda i: (0, 0)),
        ],
        out_specs=[
            pl.BlockSpec((BR // 2, D), lambda i: (i, 0)),
            pl.BlockSpec((BR, D), lambda i: (i, 0)),
        ],
        out_shape=[
            jax.ShapeDtypeStruct((N // 2, D), jnp.float32),
            jax.ShapeDtypeStruct((N, D), jnp.float32),
        ],
    )(op, st, gamma.reshape(1, D), beta.reshape(1, D), wp, bp.reshape(1, D), ws)


def _tc_out_body(s_ref, ng_ref, wn_ref, b_ref, op_ref, st_ref):
    i = pl.program_id(0)
    nb = jnp.dot(ng_ref[...], wn_ref[...].T, preferred_element_type=jnp.float32)
    o = s_ref[...] + nb + b_ref[...]
    op_ref[...] = o

    @pl.when(i == 0)
    def _():
        st_ref[...] = jnp.zeros((2, D), jnp.float32)

    ps = jnp.sum(o, axis=0, keepdims=True)
    pss = jnp.sum(o * o, axis=0, keepdims=True)
    st_ref[...] += jnp.concatenate([ps, pss], axis=0)


def _tc_out(s, neigh, wn, b):
    return pl.pallas_call(
        _tc_out_body,
        grid=(N // BR,),
        in_specs=[
            pl.BlockSpec((BR, D), lambda i: (i, 0)),
            pl.BlockSpec((BR, D), lambda i: (i, 0)),
            pl.BlockSpec((D, D), lambda i: (0, 0)),
            pl.BlockSpec((1, D), lambda i: (0, 0)),
        ],
        out_specs=[
            pl.BlockSpec((BR, D), lambda i: (i, 0)),
            pl.BlockSpec((2, D), lambda i: (0, 0)),
        ],
        out_shape=[
            jax.ShapeDtypeStruct((N, D), jnp.float32),
            jax.ShapeDtypeStruct((2, D), jnp.float32),
        ],
    )(s, neigh, wn, b.reshape(1, D))


def _tc_final_body(op_ref, st_ref, g_ref, b_ref, out_ref):
    st = st_ref[...]
    mu = st[0:1, :] / N
    var = st[1:2, :] / N - mu * mu
    out_ref[...] = _norm_elu(op_ref[...], mu, var, g_ref[...], b_ref[...])


def _tc_final(op, st, gamma, beta):
    return pl.pallas_call(
        _tc_final_body,
        grid=(N // BR,),
        in_specs=[
            pl.BlockSpec((BR, D), lambda i: (i, 0)),
            pl.BlockSpec((2, D), lambda i: (0, 0)),
            pl.BlockSpec((1, D), lambda i: (0, 0)),
            pl.BlockSpec((1, D), lambda i: (0, 0)),
        ],
        out_specs=pl.BlockSpec((BR, D), lambda i: (i, 0)),
        out_shape=jax.ShapeDtypeStruct((N, D), jnp.float32),
    )(op, st, gamma.reshape(1, D), beta.reshape(1, D))


_MESH = plsc.VectorSubcoreMesh(core_axis_name="c", subcore_axis_name="s",
                               num_cores=2, num_subcores=16)


def _wid():
    return lax.axis_index("s") * 2 + lax.axis_index("c")


def _take(x, idx):
    return lax.gather(
        x, idx[:, None],
        lax.GatherDimensionNumbers(offset_dims=(), collapsed_slice_dims=(0,),
                                   start_index_map=(0,)),
        (1,), mode=lax.GatherScatterMode.PROMISE_IN_BOUNDS)


def _bucket_body(src_hbm, dst_hbm, w_hbm,
                 bsrc, bdst, bw, bcnt,
                 src_a, dst_a, w_a, src_b, dst_b, w_b,
                 st_src, st_dst, st_w, cnt_v, sem_a, sem_b):
    wid = _wid()
    A = (src_a, dst_a, w_a)
    B = (src_b, dst_b, w_b)

    def issue(c, bufs, sem):
        cbase = pl.multiple_of(c * CHUNK, 8)
        pltpu.async_copy(src_hbm.at[pl.ds(cbase, CHUNK)], bufs[0], sem)
        pltpu.async_copy(dst_hbm.at[pl.ds(cbase, CHUNK)], bufs[1], sem)
        pltpu.async_copy(w_hbm.at[pl.ds(cbase, CHUNK)], bufs[2], sem)

    def wait(bufs, sem):
        pltpu.make_async_copy(src_hbm.at[pl.ds(0, CHUNK)], bufs[0], sem).wait()
        pltpu.make_async_copy(dst_hbm.at[pl.ds(0, CHUNK)], bufs[1], sem).wait()
        pltpu.make_async_copy(w_hbm.at[pl.ds(0, CHUNK)], bufs[2], sem).wait()

    def flush(written, cnt):
        do = cnt >= FLUSH

        @pl.when(do)
        def _():
            base = pl.multiple_of(wid * EPAD + written, 8)
            pltpu.sync_copy(st_src.at[pl.ds(0, FLUSH)],
                            bsrc.at[pl.ds(base, FLUSH)])
            pltpu.sync_copy(st_dst.at[pl.ds(0, FLUSH)],
                            bdst.at[pl.ds(base, FLUSH)])
            pltpu.sync_copy(st_w.at[pl.ds(0, FLUSH)],
                            bw.at[pl.ds(base, FLUSH)])

            def shift(j, _):
                s = pl.ds(FLUSH + j * 16, 16)
                t = pl.ds(j * 16, 16)
                st_src[t] = st_src[s]
                st_dst[t] = st_dst[s]
                st_w[t] = st_w[s]
                return 0

            lax.fori_loop(0, (STG - FLUSH) // 16, shift, 0)

        written = jnp.where(do, written + FLUSH, written)
        cnt = jnp.where(do, cnt - FLUSH, cnt)
        return written, cnt

    def compact(bufs, carry):
        written, cnt = carry
        sc, dc, wc = bufs

        @plsc.parallel_loop(0, CHUNK // 16, unroll=4,
                            carry=jnp.full((16,), cnt, jnp.int32))
        def cntv(i, cntv):
            sl = pl.ds(i * 16, 16)
            d = dc[sl]
            b = (d * MAGIC) >> MSHIFT
            m = b == wid
            sv = sc[sl]
            dl = (d - b * RPS) | ((sv & 1) << 14)
            mi = m.astype(jnp.int32)
            pref = plsc.cumsum(mi)
            pos = cntv + pref - mi
            plsc.store_scatter(st_src, [pos], sv >> 1, mask=m)
            plsc.store_scatter(st_dst, [pos], dl, mask=m)
            plsc.store_scatter(st_w, [pos], wc[sl], mask=m)
            return cntv + plsc.all_reduce_population_count(m)

        return flush(written, cntv[0])

    issue(0, A, sem_a)

    def pair_body(p, carry):
        issue(2 * p + 1, B, sem_b)
        wait(A, sem_a)
        carry = compact(A, carry)

        @pl.when(p + 1 < NCHUNKS // 2)
        def _():
            issue(2 * p + 2, A, sem_a)

        wait(B, sem_b)
        carry = compact(B, carry)
        return carry

    written, cnt = lax.fori_loop(0, NCHUNKS // 2, pair_body,
                                 (jnp.int32(0), jnp.int32(0)))

    for k in range(16):
        sl = pl.ds(cnt + k * 16, 16)
        st_src[sl] = jnp.zeros((16,), jnp.int32)
        st_dst[sl] = jnp.full((16,), SENT, jnp.int32)
        st_w[sl] = jnp.zeros((16,), jnp.float32)
    cnt = ((cnt + 255) >> 8) << 8

    written, cnt = flush(written, cnt)
    base = pl.multiple_of(wid * EPAD + written, 8)
    pltpu.sync_copy(st_src.at[pl.ds(0, FLUSH)],
                    bsrc.at[pl.ds(base, FLUSH)])
    pltpu.sync_copy(st_dst.at[pl.ds(0, FLUSH)],
                    bdst.at[pl.ds(base, FLUSH)])
    pltpu.sync_copy(st_w.at[pl.ds(0, FLUSH)],
                    bw.at[pl.ds(base, FLUSH)])
    total = written + cnt
    cnt_v[...] = jnp.full((16,), total, jnp.int32)
    pltpu.sync_copy(cnt_v, bcnt.at[pl.ds(pl.multiple_of(wid * 16, 16), 16)])


_bucket = pl.kernel(
    _bucket_body,
    out_type=(
        jax.ShapeDtypeStruct((NW * EPAD,), jnp.int32),
        jax.ShapeDtypeStruct((NW * EPAD,), jnp.int32),
        jax.ShapeDtypeStruct((NW * EPAD,), jnp.float32),
        jax.ShapeDtypeStruct((NW * 16,), jnp.int32),
    ),
    mesh=_MESH,
    compiler_params=pltpu.CompilerParams(needs_layout_passes=False),
    scratch_types=[
        pltpu.VMEM((CHUNK,), jnp.int32),
        pltpu.VMEM((CHUNK,), jnp.int32),
        pltpu.VMEM((CHUNK,), jnp.float32),
        pltpu.VMEM((CHUNK,), jnp.int32),
        pltpu.VMEM((CHUNK,), jnp.int32),
        pltpu.VMEM((CHUNK,), jnp.float32),
        pltpu.VMEM((STG,), jnp.int32),
        pltpu.VMEM((STG,), jnp.int32),
        pltpu.VMEM((STG,), jnp.float32),
        pltpu.VMEM((16,), jnp.int32),
        pltpu.SemaphoreType.DMA,
        pltpu.SemaphoreType.DMA,
    ],
)


def _segmax_body(h_hbm, bsrc, bdst, bw, bcnt,
                 out_hbm,
                 acc, msrc, mdst, mw, rows_a, rows_b, h_sp, cnt_v,
                 sem_a, sem_b):
    wid = _wid()
    iota = lax.iota(jnp.int32, 16)

    @pl.when(lax.axis_index("s") == 0)
    def _():
        pltpu.sync_copy(h_hbm, h_sp)

    plsc.subcore_barrier()


    def zero_body(r, _):
        acc[pl.ds(r * 16, 16)] = jnp.zeros((16,), jnp.float32)
        return 0

    lax.fori_loop(0, (RPS + 1) * D // 16, zero_body, 0)

    pltpu.sync_copy(bcnt.at[pl.ds(pl.multiple_of(wid * 16, 16), 16)], cnt_v)
    cnt = cnt_v[...][0]
    ebase = wid * EPAD
    nsb = (cnt + SB - 1) >> 12

    def issue(g, rows, sem):
        pltpu.async_copy(h_sp.at[msrc.at[pl.ds(g * GS, GS)]], rows, sem)

    def wait(rows, sem):
        pltpu.make_async_copy(h_sp.at[pl.ds(0, GS)], rows, sem).wait()

    def process(g, rows):
        gb = g * GS

        def blk(b, _):
            sl = pl.ds(gb + b * 16, 16)
            dvec = mdst[sl]
            wvec = mw[sl]
            ds_ = [dvec[i] for i in range(16)]
            wb_ = [_take(wvec, jnp.full((16,), i, jnp.int32)) for i in range(16)]
            for i in range(16):
                v = ds_[i]
                d = v & 16383
                roff = (v >> 14) << 6
                wb = wb_[i]
                base = d * D
                r = b * 16 + i
                for c in range(4):
                    pv = rows[r, pl.ds(roff + c * 16, 16)]
                    pb = plsc.bitcast(pv, jnp.bfloat16)
                    a, b2 = plsc.unpack(pb, format=plsc.PackFormat.INTERLEAVED)
                    fs0 = pl.ds(base + c * 32, 16)
                    fs1 = pl.ds(base + c * 32 + 16, 16)
                    acc[fs0] = jnp.maximum(acc[fs0], a * wb)
                    acc[fs1] = jnp.maximum(acc[fs1], b2 * wb)
            return 0

        lax.fori_loop(0, GS // 16, blk, 0)

    def sb_body(sb, _):
        mbase = pl.multiple_of(ebase + sb * SB, 8)
        pltpu.sync_copy(bsrc.at[pl.ds(mbase, SB)], msrc)
        pltpu.sync_copy(bdst.at[pl.ds(mbase, SB)], mdst)
        pltpu.sync_copy(bw.at[pl.ds(mbase, SB)], mw)
        rem = cnt - sb * SB
        npairs = jnp.minimum(rem, SB) >> 7

        issue(0, rows_a, sem_a)

        def pair_body(j, _):
            issue(2 * j + 1, rows_b, sem_b)
            wait(rows_a, sem_a)
            process(2 * j, rows_a)

            @pl.when(j + 1 < npairs)
            def _():
                issue(2 * j + 2, rows_a, sem_a)

            wait(rows_b, sem_b)
            process(2 * j + 1, rows_b)
            return 0

        lax.fori_loop(0, npairs, pair_body, 0)
        return 0

    lax.fori_loop(0, nsb, sb_body, 0)
    pltpu.sync_copy(acc.at[pl.ds(0, RPS * D)], out_hbm.at[wid])


_segmax = pl.kernel(
    _segmax_body,
    out_type=jax.ShapeDtypeStruct((NW, RPS * D), jnp.float32),
    mesh=_MESH,
    compiler_params=pltpu.CompilerParams(needs_layout_passes=False),
    scratch_types=[
        pltpu.VMEM(((RPS + 1) * D,), jnp.float32),
        pltpu.VMEM((SB,), jnp.int32),
        pltpu.VMEM((SB,), jnp.int32),
        pltpu.VMEM((SB,), jnp.float32),
        pltpu.VMEM((GS, D), jnp.float32),
        pltpu.VMEM((GS, D), jnp.float32),
        pltpu.VMEM_SHARED((N // 2, D), jnp.float32),
        pltpu.VMEM((16,), jnp.int32),
        pltpu.SemaphoreType.DMA,
        pltpu.SemaphoreType.DMA,
    ],
)


@jax.jit
def kernel(node_weight, edge_index, edge_weight,
           W_pool_0, b_pool_0, W_neigh_0, W_self_0, b_sage_0, gamma_0, beta_0,
           W_pool_1, b_pool_1, W_neigh_1, W_self_1, b_sage_1, gamma_1, beta_1,
           W_pool_2, b_pool_2, W_neigh_2, W_self_2, b_sage_2, gamma_2, beta_2):
    params = [
        (W_pool_0, b_pool_0, W_neigh_0, W_self_0, b_sage_0, gamma_0, beta_0),
        (W_pool_1, b_pool_1, W_neigh_1, W_self_1, b_sage_1, gamma_1, beta_1),
        (W_pool_2, b_pool_2, W_neigh_2, W_self_2, b_sage_2, gamma_2, beta_2),
    ]
    src = edge_index[0]
    dst = edge_index[1]

    bsrc, bdst, bw, bcnt = _bucket(src, dst, edge_weight)

    op, st = None, None
    for i in range(NUM_LAYERS):
        wp, bp, wn, ws, b, gamma, beta = params[i]
        if i == 0:
            h, s = _tc_in(node_weight, wp, bp, ws)
        else:
            h, s = _tc_in_fused(op, st, params[i - 1][5], params[i - 1][6],
                                wp, bp, ws)
        neigh = _segmax(h, bsrc, bdst, bw, bcnt)
        neigh = neigh.reshape(NW * RPS, D)[:N]
        op, st = _tc_out(s, neigh, wn, b)

    return _tc_final(op, st, params[2][5], params[2][6])


# speedup vs baseline: 9.9936x; 1.3801x over previous
import functools
import jax
import jax.numpy as jnp
from jax import lax
from jax.experimental import pallas as pl
from jax.experimental.pallas import tpu as pltpu
from jax.experimental.pallas import tpu_sc as plsc

N = 10000
E = 320000
D = 128
NUM_LAYERS = 3
EPS = 1e-5

NW = 32
RPS = 313
SENT = RPS
MAGIC = 13401
MSHIFT = 22

CHUNK = 2000
NCHUNKS = E // CHUNK
STG = 4096
FLUSH = 2048
GS = 64
SB = 4096
EPAD = E + SB + 256

BR = 2000


def _pack_feats(hb):
    parts = []
    for c in range(4):
        lo = hb[:, c * 32:c * 32 + 16].astype(jnp.uint32)
        hi = hb[:, c * 32 + 16:c * 32 + 32].astype(jnp.uint32)
        parts.append(lo | (hi << 16))
    return jnp.concatenate(parts, axis=1)


def _pack_rows(h):
    hb = lax.bitcast_convert_type(h.astype(jnp.bfloat16), jnp.uint16)
    hb3 = hb.reshape(h.shape[0] // 2, 2, 128)
    a = _pack_feats(hb3[:, 0, :])
    b = _pack_feats(hb3[:, 1, :])
    return lax.bitcast_convert_type(jnp.concatenate([a, b], axis=1),
                                    jnp.float32)


def _tc_in_body(x_ref, wp_ref, bp_ref, ws_ref, h_ref, s_ref):
    x = x_ref[...]
    h = jnp.dot(x, wp_ref[...].T, preferred_element_type=jnp.float32)
    h_ref[...] = _pack_rows(jnp.maximum(h + bp_ref[...], 0.0))
    s_ref[...] = jnp.dot(x, ws_ref[...].T, preferred_element_type=jnp.float32)


def _tc_in(x, wp, bp, ws):
    return pl.pallas_call(
        _tc_in_body,
        grid=(N // BR,),
        in_specs=[
            pl.BlockSpec((BR, D), lambda i: (i, 0)),
            pl.BlockSpec((D, D), lambda i: (0, 0)),
            pl.BlockSpec((1, D), lambda i: (0, 0)),
            pl.BlockSpec((D, D), lambda i: (0, 0)),
        ],
        out_specs=[
            pl.BlockSpec((BR // 2, D), lambda i: (i, 0)),
            pl.BlockSpec((BR, D), lambda i: (i, 0)),
        ],
        out_shape=[
            jax.ShapeDtypeStruct((N // 2, D), jnp.float32),
            jax.ShapeDtypeStruct((N, D), jnp.float32),
        ],
    )(x, wp, bp.reshape(1, D), ws)


def _norm_elu(op, mu, var, gamma, beta):
    inv = lax.rsqrt(var + EPS)
    xn = (op - mu) * inv * gamma + beta
    return jnp.where(xn > 0.0, xn, jnp.exp(jnp.minimum(xn, 0.0)) - 1.0)


def _tc_in_fused_body(op_ref, st_ref, g_ref, b_ref, wp_ref, bp_ref, ws_ref,
                      h_ref, s_ref):
    st = st_ref[...]
    mu = st[0:1, :] / N
    var = st[1:2, :] / N - mu * mu
    x = _norm_elu(op_ref[...], mu, var, g_ref[...], b_ref[...])
    h = jnp.dot(x, wp_ref[...].T, preferred_element_type=jnp.float32)
    h_ref[...] = _pack_rows(jnp.maximum(h + bp_ref[...], 0.0))
    s_ref[...] = jnp.dot(x, ws_ref[...].T, preferred_element_type=jnp.float32)


def _tc_in_fused(op, st, gamma, beta, wp, bp, ws):
    return pl.pallas_call(
        _tc_in_fused_body,
        grid=(N // BR,),
        in_specs=[
            pl.BlockSpec((BR, D), lambda i: (i, 0)),
            pl.BlockSpec((2, D), lambda i: (0, 0)),
            pl.BlockSpec((1, D), lambda i: (0, 0)),
            pl.BlockSpec((1, D), lambda i: (0, 0)),
            pl.BlockSpec((D, D), lambda i: (0, 0)),
            pl.BlockSpec((1, D), lambda i: (0, 0)),
            pl.BlockSpec((D, D), lambda i: (0, 0)),
        ],
        out_specs=[
            pl.BlockSpec((BR // 2, D), lambda i: (i, 0)),
            pl.BlockSpec((BR, D), lambda i: (i, 0)),
        ],
        out_shape=[
            jax.ShapeDtypeStruct((N // 2, D), jnp.float32),
            jax.ShapeDtypeStruct((N, D), jnp.float32),
        ],
    )(op, st, gamma.reshape(1, D), beta.reshape(1, D), wp, bp.reshape(1, D), ws)


def _tc_out_body(s_ref, ng_ref, wn_ref, b_ref, op_ref, st_ref):
    i = pl.program_id(0)
    nb = jnp.dot(ng_ref[...], wn_ref[...].T, preferred_element_type=jnp.float32)
    o = s_ref[...] + nb + b_ref[...]
    op_ref[...] = o

    @pl.when(i == 0)
    def _():
        st_ref[...] = jnp.zeros((2, D), jnp.float32)

    ps = jnp.sum(o, axis=0, keepdims=True)
    pss = jnp.sum(o * o, axis=0, keepdims=True)
    st_ref[...] += jnp.concatenate([ps, pss], axis=0)


def _tc_out(s, neigh, wn, b):
    return pl.pallas_call(
        _tc_out_body,
        grid=(N // BR,),
        in_specs=[
            pl.BlockSpec((BR, D), lambda i: (i, 0)),
            pl.BlockSpec((BR, D), lambda i: (i, 0)),
            pl.BlockSpec((D, D), lambda i: (0, 0)),
            pl.BlockSpec((1, D), lambda i: (0, 0)),
        ],
        out_specs=[
            pl.BlockSpec((BR, D), lambda i: (i, 0)),
            pl.BlockSpec((2, D), lambda i: (0, 0)),
        ],
        out_shape=[
            jax.ShapeDtypeStruct((N, D), jnp.float32),
            jax.ShapeDtypeStruct((2, D), jnp.float32),
        ],
    )(s, neigh, wn, b.reshape(1, D))


def _tc_final_body(op_ref, st_ref, g_ref, b_ref, out_ref):
    st = st_ref[...]
    mu = st[0:1, :] / N
    var = st[1:2, :] / N - mu * mu
    out_ref[...] = _norm_elu(op_ref[...], mu, var, g_ref[...], b_ref[...])


def _tc_final(op, st, gamma, beta):
    return pl.pallas_call(
        _tc_final_body,
        grid=(N // BR,),
        in_specs=[
            pl.BlockSpec((BR, D), lambda i: (i, 0)),
            pl.BlockSpec((2, D), lambda i: (0, 0)),
            pl.BlockSpec((1, D), lambda i: (0, 0)),
            pl.BlockSpec((1, D), lambda i: (0, 0)),
        ],
        out_specs=pl.BlockSpec((BR, D), lambda i: (i, 0)),
        out_shape=jax.ShapeDtypeStruct((N, D), jnp.float32),
    )(op, st, gamma.reshape(1, D), beta.reshape(1, D))


_MESH = plsc.VectorSubcoreMesh(core_axis_name="c", subcore_axis_name="s",
                               num_cores=2, num_subcores=16)


def _wid():
    return lax.axis_index("s") * 2 + lax.axis_index("c")


def _take(x, idx):
    return lax.gather(
        x, idx[:, None],
        lax.GatherDimensionNumbers(offset_dims=(), collapsed_slice_dims=(0,),
                                   start_index_map=(0,)),
        (1,), mode=lax.GatherScatterMode.PROMISE_IN_BOUNDS)


def _bucket_body(src_hbm, dst_hbm, w_hbm,
                 bsrc, bdst, bw, bcnt,
                 src_a, dst_a, w_a, src_b, dst_b, w_b,
                 st_src, st_dst, st_w, cnt_v, sem_a, sem_b):
    wid = _wid()
    A = (src_a, dst_a, w_a)
    B = (src_b, dst_b, w_b)

    def issue(c, bufs, sem):
        cbase = pl.multiple_of(c * CHUNK, 8)
        pltpu.async_copy(src_hbm.at[pl.ds(cbase, CHUNK)], bufs[0], sem)
        pltpu.async_copy(dst_hbm.at[pl.ds(cbase, CHUNK)], bufs[1], sem)
        pltpu.async_copy(w_hbm.at[pl.ds(cbase, CHUNK)], bufs[2], sem)

    def wait(bufs, sem):
        pltpu.make_async_copy(src_hbm.at[pl.ds(0, CHUNK)], bufs[0], sem).wait()
        pltpu.make_async_copy(dst_hbm.at[pl.ds(0, CHUNK)], bufs[1], sem).wait()
        pltpu.make_async_copy(w_hbm.at[pl.ds(0, CHUNK)], bufs[2], sem).wait()

    def flush(written, cnt):
        do = cnt >= FLUSH

        @pl.when(do)
        def _():
            base = pl.multiple_of(wid * EPAD + written, 8)
            pltpu.sync_copy(st_src.at[pl.ds(0, FLUSH)],
                            bsrc.at[pl.ds(base, FLUSH)])
            pltpu.sync_copy(st_dst.at[pl.ds(0, FLUSH)],
                            bdst.at[pl.ds(base, FLUSH)])
            pltpu.sync_copy(st_w.at[pl.ds(0, FLUSH)],
                            bw.at[pl.ds(base, FLUSH)])

            def shift(j, _):
                s = pl.ds(FLUSH + j * 16, 16)
                t = pl.ds(j * 16, 16)
                st_src[t] = st_src[s]
                st_dst[t] = st_dst[s]
                st_w[t] = st_w[s]
                return 0

            lax.fori_loop(0, (STG - FLUSH) // 16, shift, 0)

        written = jnp.where(do, written + FLUSH, written)
        cnt = jnp.where(do, cnt - FLUSH, cnt)
        return written, cnt

    def compact(bufs, carry):
        written, cnt = carry
        sc, dc, wc = bufs

        @plsc.parallel_loop(0, CHUNK // 16, unroll=4,
                            carry=jnp.full((16,), cnt, jnp.int32))
        def cntv(i, cntv):
            sl = pl.ds(i * 16, 16)
            d = dc[sl]
            b = (d * MAGIC) >> MSHIFT
            m = b == wid
            sv = sc[sl]
            dl = (d - b * RPS) | ((sv & 1) << 14)
            mi = m.astype(jnp.int32)
            pref = plsc.cumsum(mi)
            pos = cntv + pref - mi
            plsc.store_scatter(st_src, [pos], sv >> 1, mask=m)
            plsc.store_scatter(st_dst, [pos], dl, mask=m)
            plsc.store_scatter(st_w, [pos], wc[sl], mask=m)
            return cntv + plsc.all_reduce_population_count(m)

        return flush(written, cntv[0])

    issue(0, A, sem_a)

    def pair_body(p, carry):
        issue(2 * p + 1, B, sem_b)
        wait(A, sem_a)
        carry = compact(A, carry)

        @pl.when(p + 1 < NCHUNKS // 2)
        def _():
            issue(2 * p + 2, A, sem_a)

        wait(B, sem_b)
        carry = compact(B, carry)
        return carry

    written, cnt = lax.fori_loop(0, NCHUNKS // 2, pair_body,
                                 (jnp.int32(0), jnp.int32(0)))

    for k in range(16):
        sl = pl.ds(cnt + k * 16, 16)
        st_src[sl] = jnp.zeros((16,), jnp.int32)
        st_dst[sl] = jnp.full((16,), SENT, jnp.int32)
        st_w[sl] = jnp.zeros((16,), jnp.float32)
    cnt = ((cnt + 255) >> 8) << 8

    written, cnt = flush(written, cnt)
    base = pl.multiple_of(wid * EPAD + written, 8)
    pltpu.sync_copy(st_src.at[pl.ds(0, FLUSH)],
                    bsrc.at[pl.ds(base, FLUSH)])
    pltpu.sync_copy(st_dst.at[pl.ds(0, FLUSH)],
                    bdst.at[pl.ds(base, FLUSH)])
    pltpu.sync_copy(st_w.at[pl.ds(0, FLUSH)],
                    bw.at[pl.ds(base, FLUSH)])
    total = written + cnt
    cnt_v[...] = jnp.full((16,), total, jnp.int32)
    pltpu.sync_copy(cnt_v, bcnt.at[pl.ds(pl.multiple_of(wid * 16, 16), 16)])


_bucket = pl.kernel(
    _bucket_body,
    out_type=(
        jax.ShapeDtypeStruct((NW * EPAD,), jnp.int32),
        jax.ShapeDtypeStruct((NW * EPAD,), jnp.int32),
        jax.ShapeDtypeStruct((NW * EPAD,), jnp.float32),
        jax.ShapeDtypeStruct((NW * 16,), jnp.int32),
    ),
    mesh=_MESH,
    compiler_params=pltpu.CompilerParams(needs_layout_passes=False),
    scratch_types=[
        pltpu.VMEM((CHUNK,), jnp.int32),
        pltpu.VMEM((CHUNK,), jnp.int32),
        pltpu.VMEM((CHUNK,), jnp.float32),
        pltpu.VMEM((CHUNK,), jnp.int32),
        pltpu.VMEM((CHUNK,), jnp.int32),
        pltpu.VMEM((CHUNK,), jnp.float32),
        pltpu.VMEM((STG,), jnp.int32),
        pltpu.VMEM((STG,), jnp.int32),
        pltpu.VMEM((STG,), jnp.float32),
        pltpu.VMEM((16,), jnp.int32),
        pltpu.SemaphoreType.DMA,
        pltpu.SemaphoreType.DMA,
    ],
)


def _segmax_body(h_hbm, bsrc, bdst, bw, bcnt,
                 out_hbm,
                 acc, msrc, mdst, mw, rows_a, rows_b, h_sp, cnt_v,
                 sem_a, sem_b):
    wid = _wid()
    iota = lax.iota(jnp.int32, 16)

    @pl.when(lax.axis_index("s") == 0)
    def _():
        pltpu.sync_copy(h_hbm, h_sp)

    plsc.subcore_barrier()


    def zero_body(r, _):
        acc[pl.ds(r * 16, 16)] = jnp.zeros((16,), jnp.float32)
        return 0

    lax.fori_loop(0, (RPS + 1) * D // 16, zero_body, 0)

    pltpu.sync_copy(bcnt.at[pl.ds(pl.multiple_of(wid * 16, 16), 16)], cnt_v)
    cnt = cnt_v[...][0]
    ebase = wid * EPAD
    nsb = (cnt + SB - 1) >> 12

    def issue(g, rows, sem):
        pltpu.async_copy(h_sp.at[msrc.at[pl.ds(g * GS, GS)]], rows, sem)

    def wait(rows, sem):
        pltpu.make_async_copy(h_sp.at[pl.ds(0, GS)], rows, sem).wait()

    def process(g, rows):
        gb = g * GS

        def blk(b, _):
            sl = pl.ds(gb + b * 16, 16)
            dvec = mdst[sl]
            wvec = mw[sl]
            srt, _p = plsc.sort_key_val(dvec, iota)
            prev = _take(srt, jnp.maximum(iota - 1, 0))
            dupm = (srt == prev) & (iota > 0)
            has_dup = plsc.all_reduce_population_count(dupm)[0] > 0

            def edge_update(v, wb, r):
                d = v & 16383
                roff = (v >> 14) << 6
                base = d * D
                for c in range(4):
                    pv = rows[r, pl.ds(roff + c * 16, 16)]
                    pb = plsc.bitcast(pv, jnp.bfloat16)
                    a, b2 = plsc.unpack(pb, format=plsc.PackFormat.INTERLEAVED)
                    fs0 = pl.ds(base + c * 32, 16)
                    fs1 = pl.ds(base + c * 32 + 16, 16)
                    acc[fs0] = jnp.maximum(acc[fs0], a * wb)
                    acc[fs1] = jnp.maximum(acc[fs1], b2 * wb)

            @pl.when(jnp.logical_not(has_dup))
            def _():
                @plsc.parallel_loop(0, 16, unroll=2)
                def _(i):
                    sel = jnp.full((16,), i, jnp.int32)
                    v = _take(dvec, sel)[0]
                    wb = _take(wvec, sel)
                    edge_update(v, wb, b * 16 + i)

            @pl.when(has_dup)
            def _():
                ds_ = [dvec[i] for i in range(16)]
                wb_ = [_take(wvec, jnp.full((16,), i, jnp.int32))
                       for i in range(16)]
                for i in range(16):
                    edge_update(ds_[i], wb_[i], b * 16 + i)
            return 0

        lax.fori_loop(0, GS // 16, blk, 0)

    def sb_body(sb, _):
        mbase = pl.multiple_of(ebase + sb * SB, 8)
        pltpu.sync_copy(bsrc.at[pl.ds(mbase, SB)], msrc)
        pltpu.sync_copy(bdst.at[pl.ds(mbase, SB)], mdst)
        pltpu.sync_copy(bw.at[pl.ds(mbase, SB)], mw)
        rem = cnt - sb * SB
        npairs = jnp.minimum(rem, SB) >> 7

        issue(0, rows_a, sem_a)

        def pair_body(j, _):
            issue(2 * j + 1, rows_b, sem_b)
            wait(rows_a, sem_a)
            process(2 * j, rows_a)

            @pl.when(j + 1 < npairs)
            def _():
                issue(2 * j + 2, rows_a, sem_a)

            wait(rows_b, sem_b)
            process(2 * j + 1, rows_b)
            return 0

        lax.fori_loop(0, npairs, pair_body, 0)
        return 0

    lax.fori_loop(0, nsb, sb_body, 0)
    pltpu.sync_copy(acc.at[pl.ds(0, RPS * D)], out_hbm.at[wid])


_segmax = pl.kernel(
    _segmax_body,
    out_type=jax.ShapeDtypeStruct((NW, RPS * D), jnp.float32),
    mesh=_MESH,
    compiler_params=pltpu.CompilerParams(needs_layout_passes=False),
    scratch_types=[
        pltpu.VMEM(((RPS + 1) * D,), jnp.float32),
        pltpu.VMEM((SB,), jnp.int32),
        pltpu.VMEM((SB,), jnp.int32),
        pltpu.VMEM((SB,), jnp.float32),
        pltpu.VMEM((GS, D), jnp.float32),
        pltpu.VMEM((GS, D), jnp.float32),
        pltpu.VMEM_SHARED((N // 2, D), jnp.float32),
        pltpu.VMEM((16,), jnp.int32),
        pltpu.SemaphoreType.DMA,
        pltpu.SemaphoreType.DMA,
    ],
)


@jax.jit
def kernel(node_weight, edge_index, edge_weight,
           W_pool_0, b_pool_0, W_neigh_0, W_self_0, b_sage_0, gamma_0, beta_0,
           W_pool_1, b_pool_1, W_neigh_1, W_self_1, b_sage_1, gamma_1, beta_1,
           W_pool_2, b_pool_2, W_neigh_2, W_self_2, b_sage_2, gamma_2, beta_2):
    params = [
        (W_pool_0, b_pool_0, W_neigh_0, W_self_0, b_sage_0, gamma_0, beta_0),
        (W_pool_1, b_pool_1, W_neigh_1, W_self_1, b_sage_1, gamma_1, beta_1),
        (W_pool_2, b_pool_2, W_neigh_2, W_self_2, b_sage_2, gamma_2, beta_2),
    ]
    src = edge_index[0]
    dst = edge_index[1]

    bsrc, bdst, bw, bcnt = _bucket(src, dst, edge_weight)

    op, st = None, None
    for i in range(NUM_LAYERS):
        wp, bp, wn, ws, b, gamma, beta = params[i]
        if i == 0:
            h, s = _tc_in(node_weight, wp, bp, ws)
        else:
            h, s = _tc_in_fused(op, st, params[i - 1][5], params[i - 1][6],
                                wp, bp, ws)
        neigh = _segmax(h, bsrc, bdst, bw, bcnt)
        neigh = neigh.reshape(NW * RPS, D)[:N]
        op, st = _tc_out(s, neigh, wn, b)

    return _tc_final(op, st, params[2][5], params[2][6])
